# Initial kernel scaffold; baseline (speedup 1.0000x reference)
import jax
import jax.numpy as jnp
from jax.experimental import pallas as pl


def kernel(x, table):
    raise NotImplementedError("write your pallas kernel here")


# speedup vs baseline: 2.9666x; 2.9666x over previous
import functools

import jax
import jax.numpy as jnp
from jax import lax
from jax.experimental import pallas as pl
from jax.experimental.pallas import tpu as pltpu
from jax.experimental.pallas import tpu_sc as plsc

D = 128
CHUNK = 128


@functools.cache
def _build(B):
    info = plsc.get_sparse_core_info()
    nw = info.num_cores * info.num_subcores
    per_w = B // nw
    n_chunks = per_w // CHUNK

    mesh = plsc.VectorSubcoreMesh(core_axis_name="c", subcore_axis_name="s")

    @functools.partial(
        pl.kernel,
        mesh=mesh,
        out_type=jax.ShapeDtypeStruct((B, D), jnp.float32),
        scratch_types=[
            pltpu.VMEM((n_chunks, CHUNK), jnp.int32),
            pltpu.VMEM((CHUNK, D), jnp.float32),
            pltpu.SemaphoreType.DMA,
        ],
    )
    def gather_kernel(x_hbm, table_hbm, out_hbm, idx_v, rows_v, sem):
        wid = lax.axis_index("s") * info.num_cores + lax.axis_index("c")
        pltpu.sync_copy(x_hbm.at[wid], idx_v)

        def body(j, carry):
            pltpu.async_copy(table_hbm.at[idx_v.at[j]], rows_v, sem).wait()
            pltpu.sync_copy(
                rows_v, out_hbm.at[pl.ds(wid * per_w + j * CHUNK, CHUNK)]
            )
            return carry

        lax.fori_loop(0, n_chunks, body, 0)

    return gather_kernel


def kernel(x, table):
    b0, b1 = x.shape
    B = b0 * b1
    info = plsc.get_sparse_core_info()
    nw = info.num_cores * info.num_subcores
    x2 = x.reshape(nw, B // (nw * CHUNK), CHUNK).astype(jnp.int32)
    out = _build(B)(x2, table)
    return out.reshape(b0, b1, D)


# speedup vs baseline: 3.3476x; 1.1284x over previous
import functools

import jax
import jax.numpy as jnp
from jax import lax
from jax.experimental import pallas as pl
from jax.experimental.pallas import tpu as pltpu
from jax.experimental.pallas import tpu_sc as plsc

D = 128
CHUNK = 128


@functools.cache
def _build(B):
    info = plsc.get_sparse_core_info()
    nw = info.num_cores * info.num_subcores
    per_w = B // nw
    n_chunks = per_w // CHUNK

    mesh = plsc.VectorSubcoreMesh(core_axis_name="c", subcore_axis_name="s")

    nb = 5
    n_rounds = n_chunks // nb

    @functools.partial(
        pl.kernel,
        mesh=mesh,
        out_type=jax.ShapeDtypeStruct((B, D), jnp.float32),
        scratch_types=[
            pltpu.VMEM((n_chunks, CHUNK), jnp.int32),
            pltpu.VMEM((nb, CHUNK, D), jnp.float32),
            pltpu.SemaphoreType.DMA,
            pltpu.SemaphoreType.DMA,
        ],
    )
    def gather_kernel(x_hbm, table_hbm, out_hbm, idx_v, rows_v, gsem, ssem):
        wid = lax.axis_index("s") * info.num_cores + lax.axis_index("c")
        out_base = wid * per_w
        pltpu.sync_copy(x_hbm.at[wid], idx_v)

        def gather(b, chunk):
            return pltpu.make_async_copy(
                table_hbm.at[idx_v.at[chunk]], rows_v.at[b], gsem
            )

        def store(b, chunk):
            return pltpu.make_async_copy(
                rows_v.at[b],
                out_hbm.at[pl.ds(out_base + chunk * CHUNK, CHUNK)],
                ssem,
            )

        for b in range(nb):
            gather(b, b).start()

        def round_body(jo, carry):
            base = jo * nb
            for b in range(nb):
                gather(b, base + b).wait()
                store(b, base + b).start()
            for b in range(nb):
                store(b, base + b).wait()
                gather(b, base + b + nb).start()
            return carry

        lax.fori_loop(0, n_rounds - 1, round_body, 0)

        last = (n_rounds - 1) * nb
        for b in range(nb):
            gather(b, last + b).wait()
            store(b, last + b).start()
        for b in range(nb):
            store(b, last + b).wait()

    return gather_kernel


def kernel(x, table):
    b0, b1 = x.shape
    B = b0 * b1
    info = plsc.get_sparse_core_info()
    nw = info.num_cores * info.num_subcores
    x2 = x.reshape(nw, B // (nw * CHUNK), CHUNK).astype(jnp.int32)
    out = _build(B)(x2, table)
    return out.reshape(b0, b1, D)


# speedup vs baseline: 3.3477x; 1.0000x over previous
import functools

import jax
import jax.numpy as jnp
from jax import lax
from jax.experimental import pallas as pl
from jax.experimental.pallas import tpu as pltpu
from jax.experimental.pallas import tpu_sc as plsc

D = 128
CHUNK = 128


@functools.cache
def _build(B):
    info = plsc.get_sparse_core_info()
    nw = info.num_cores * info.num_subcores
    per_w = B // nw
    n_chunks = per_w // CHUNK

    mesh = plsc.VectorSubcoreMesh(core_axis_name="c", subcore_axis_name="s")

    nb = 5
    k = 2

    @functools.partial(
        pl.kernel,
        mesh=mesh,
        out_type=jax.ShapeDtypeStruct((B, D), jnp.float32),
        scratch_types=[
            pltpu.VMEM((n_chunks, CHUNK), jnp.int32),
            pltpu.VMEM((nb, CHUNK, D), jnp.float32),
            pltpu.SemaphoreType.DMA((nb,)),
            pltpu.SemaphoreType.DMA((nb,)),
        ],
    )
    def gather_kernel(x_hbm, table_hbm, out_hbm, idx_v, rows_v, gsem, ssem):
        wid = lax.axis_index("s") * info.num_cores + lax.axis_index("c")
        out_base = wid * per_w
        pltpu.sync_copy(x_hbm.at[wid], idx_v)

        def gather(chunk):
            b = lax.rem(chunk, nb) if not isinstance(chunk, int) else chunk % nb
            return pltpu.make_async_copy(
                table_hbm.at[idx_v.at[chunk]], rows_v.at[b], gsem.at[b]
            )

        def store(chunk):
            b = lax.rem(chunk, nb) if not isinstance(chunk, int) else chunk % nb
            return pltpu.make_async_copy(
                rows_v.at[b],
                out_hbm.at[pl.ds(out_base + chunk * CHUNK, CHUNK)],
                ssem.at[b],
            )

        for c in range(nb):
            gather(c).start()
        for c in range(k):
            gather(c).wait()
            store(c).start()

        def body(c, carry):
            gather(c).wait()
            store(c).start()
            d = c - k
            store(d).wait()
            gather(d + nb).start()
            return carry

        lax.fori_loop(k, n_chunks - (nb - k), body, 0)

        for c in range(n_chunks - (nb - k), n_chunks):
            gather(c).wait()
            store(c).start()
        for c in range(n_chunks - nb, n_chunks):
            store(c).wait()

    return gather_kernel


def kernel(x, table):
    b0, b1 = x.shape
    B = b0 * b1
    info = plsc.get_sparse_core_info()
    nw = info.num_cores * info.num_subcores
    x2 = x.reshape(nw, B // (nw * CHUNK), CHUNK).astype(jnp.int32)
    out = _build(B)(x2, table)
    return out.reshape(b0, b1, D)


# speedup vs baseline: 5.9878x; 1.7886x over previous
import functools

import jax
import jax.numpy as jnp
from jax import lax
from jax.experimental import pallas as pl
from jax.experimental.pallas import tpu as pltpu
from jax.experimental.pallas import tpu_sc as plsc

D = 128


@functools.cache
def _build(n_samples, seq):
    info = plsc.get_sparse_core_info()
    nw = info.num_cores * info.num_subcores
    per_w = n_samples // nw
    nb = 8
    k = 3

    mesh = plsc.VectorSubcoreMesh(core_axis_name="c", subcore_axis_name="s")

    @functools.partial(
        pl.kernel,
        mesh=mesh,
        out_type=jax.ShapeDtypeStruct((n_samples, seq, D), jnp.float32),
        scratch_types=[
            pltpu.VMEM((per_w, seq), jnp.int32),
            pltpu.VMEM((nb, seq, D), jnp.float32),
            pltpu.SemaphoreType.DMA((nb,)),
            pltpu.SemaphoreType.DMA((nb,)),
        ],
    )
    def gather_kernel(x_hbm, table_hbm, out_hbm, idx_v, rows_v, gsem, ssem):
        wid = lax.axis_index("s") * info.num_cores + lax.axis_index("c")
        s_base = wid * per_w
        pltpu.sync_copy(x_hbm.at[pl.ds(s_base, per_w)], idx_v)

        def buf(i):
            return i % nb if isinstance(i, int) else lax.rem(i, nb)

        def gather(i):
            b = buf(i)
            return pltpu.make_async_copy(
                table_hbm.at[idx_v.at[i]], rows_v.at[b], gsem.at[b]
            )

        def store(i):
            b = buf(i)
            return pltpu.make_async_copy(
                rows_v.at[b], out_hbm.at[s_base + i], ssem.at[b]
            )

        for i in range(nb):
            gather(i).start()
        for i in range(k):
            gather(i).wait()
            store(i).start()

        def body(i, carry):
            gather(i).wait()
            store(i).start()
            d = i - k
            store(d).wait()
            gather(d + nb).start()
            return carry

        lax.fori_loop(k, per_w - (nb - k), body, 0)

        for i in range(per_w - (nb - k), per_w):
            gather(i).wait()
            store(i).start()
        for i in range(per_w - nb, per_w):
            store(i).wait()

    return gather_kernel


def kernel(x, table):
    n_samples, seq = x.shape
    return _build(n_samples, seq)(x.astype(jnp.int32), table)
